# baseline (speedup 1.0000x reference)
import functools

import jax
import jax.numpy as jnp
from jax import lax
from jax.experimental import pallas as pl
from jax.experimental.pallas import tpu as pltpu
from jax.experimental.pallas import tpu_sc as plsc

VOCAB = 1000000
BATCH = 4096
SEQ = 200
HIDDEN = 64
NUM_LABELS = 128

NUM_CORES = 2
NUM_SUBCORES = 16
NUM_WORKERS = NUM_CORES * NUM_SUBCORES
LANES = 16

ROWS_PER_WORKER = BATCH // NUM_WORKERS
CHUNK0 = 128
CHUNK1 = SEQ - CHUNK0


BI = 8192


def _project_body(t_ref, w_ref, b_ref, o_ref):
    logits = (
        lax.dot_general(
            t_ref[...],
            w_ref[...],
            (((0,), (1,)), ((), ())),
            preferred_element_type=jnp.float32,
        )
        + b_ref[...]
    )

    def pk(rows):
        lo = lax.convert_element_type(
            lax.bitcast_convert_type(
                lax.convert_element_type(rows[:, :64], jnp.bfloat16),
                jnp.uint16,
            ),
            jnp.uint32,
        )
        hi = lax.convert_element_type(
            lax.bitcast_convert_type(
                lax.convert_element_type(rows[:, 64:], jnp.bfloat16),
                jnp.uint16,
            ),
            jnp.uint32,
        )
        return lax.bitcast_convert_type(lo | (hi << 16), jnp.int32)

    o_ref[:, :64] = pk(logits[: BI // 2])
    o_ref[:, 64:] = pk(logits[BI // 2 :])


NBLOCKS = (VOCAB + BI - 1) // BI
VOCAB2 = NBLOCKS * (BI // 2)


def _project(tableT, W, b2d):
    return pl.pallas_call(
        _project_body,
        grid=(NBLOCKS,),
        in_specs=[
            pl.BlockSpec((HIDDEN, BI), lambda i: (0, i)),
            pl.BlockSpec((NUM_LABELS, HIDDEN), lambda i: (0, 0)),
            pl.BlockSpec((1, NUM_LABELS), lambda i: (0, 0)),
        ],
        out_specs=pl.BlockSpec((BI // 2, NUM_LABELS), lambda i: (i, 0)),
        out_shape=jax.ShapeDtypeStruct((VOCAB2, NUM_LABELS), jnp.int32),
    )(tableT, W, b2d)


def _pool_body(
    idx_hbm, par_hbm, table_hbm, out_hbm, idx_v, par_v, rows_v, out_v, sem0, sem1
):
    wid = lax.axis_index("s") * NUM_CORES + lax.axis_index("c")
    base_row = wid * ROWS_PER_WORKER

    pltpu.sync_copy(
        idx_hbm.at[pl.ds(base_row * SEQ, ROWS_PER_WORKER * SEQ)], idx_v
    )
    pltpu.sync_copy(
        par_hbm.at[pl.ds(base_row * SEQ, ROWS_PER_WORKER * SEQ)],
        par_v.at[pl.ds(0, ROWS_PER_WORKER * SEQ)],
    )

    sems = (sem0, sem1)

    def gather_descs(r, buf):
        off0 = pl.multiple_of(r * SEQ, 8)
        off1 = pl.multiple_of(r * SEQ + CHUNK0, 8)
        return (
            (
                table_hbm.at[idx_v.at[pl.ds(off0, CHUNK0)]],
                rows_v.at[buf].at[pl.ds(0, CHUNK0)],
                sems[buf],
            ),
            (
                table_hbm.at[idx_v.at[pl.ds(off1, CHUNK1)]],
                rows_v.at[buf].at[pl.ds(CHUNK0, CHUNK1)],
                sems[buf],
            ),
        )

    def start(r, buf):
        for desc in gather_descs(r, buf):
            pltpu.async_copy(*desc)

    def wait(r, buf):
        for desc in gather_descs(r, buf):
            pltpu.make_async_copy(*desc).wait()

    def accum(r, buf):
        def g_body(g, acc):
            par_vec = par_v[pl.ds(r * SEQ + g * 8, LANES)]
            acc = list(acc)
            for k in range(8):
                s = g * 8 + k
                base = par_vec[k]
                for j in range(4):
                    w = rows_v[buf, s, pl.ds(base + j * LANES, LANES)]
                    acc[j] = acc[j] + plsc.bitcast(w << 16, jnp.float32)
                    acc[4 + j] = acc[4 + j] + plsc.bitcast(
                        w & jnp.int32(-65536), jnp.float32
                    )
            return tuple(acc)

        zero = jnp.zeros((LANES,), jnp.float32)
        acc = lax.fori_loop(0, SEQ // 8, g_body, (zero,) * 8)
        for j in range(8):
            out_v[r, pl.ds(j * LANES, LANES)] = acc[j]

    start(0, 0)

    def pair_body(k, carry):
        r0 = k * 2
        start(r0 + 1, 1)
        wait(r0, 0)
        accum(r0, 0)

        @pl.when(k < ROWS_PER_WORKER // 2 - 1)
        def _():
            start(r0 + 2, 0)

        wait(r0 + 1, 1)
        accum(r0 + 1, 1)
        return carry

    lax.fori_loop(0, ROWS_PER_WORKER // 2, pair_body, 0)
    pltpu.sync_copy(out_v, out_hbm.at[pl.ds(base_row, ROWS_PER_WORKER)])


_pool = pl.kernel(
    _pool_body,
    out_type=jax.ShapeDtypeStruct((BATCH, NUM_LABELS), jnp.float32),
    mesh=plsc.VectorSubcoreMesh(
        core_axis_name="c", subcore_axis_name="s", num_cores=NUM_CORES
    ),
    scratch_types=[
        pltpu.VMEM((ROWS_PER_WORKER * SEQ,), jnp.int32),
        pltpu.VMEM((ROWS_PER_WORKER * SEQ + 16,), jnp.int32),
        pltpu.VMEM((2, SEQ, NUM_LABELS), jnp.int32),
        pltpu.VMEM((ROWS_PER_WORKER, NUM_LABELS), jnp.float32),
        pltpu.SemaphoreType.DMA,
        pltpu.SemaphoreType.DMA,
    ],
    compiler_params=pltpu.CompilerParams(use_tc_tiling_on_sc=True, needs_layout_passes=False),
)


def _head_body(x_ref, o_ref):
    logits = x_ref[...] * (1.0 / SEQ)
    m = jnp.max(logits, axis=1, keepdims=True)
    e = jnp.exp(logits - m)
    s = jnp.sum(e, axis=1, keepdims=True)
    o_ref[...] = (logits - m) - jnp.log(s)


def _head(summed):
    return pl.pallas_call(
        _head_body,
        grid=(4,),
        in_specs=[pl.BlockSpec((BATCH // 4, NUM_LABELS), lambda i: (i, 0))],
        out_specs=pl.BlockSpec((BATCH // 4, NUM_LABELS), lambda i: (i, 0)),
        out_shape=jax.ShapeDtypeStruct((BATCH, NUM_LABELS), jnp.float32),
    )(summed)


@jax.jit
def kernel(one_hot_sentence, emb_table, W, b):
    idx = one_hot_sentence.reshape(-1).astype(jnp.int32)
    blk = idx // BI
    k = idx % BI
    idx2 = blk * (BI // 2) + (k % (BI // 2))
    par = jnp.where(k >= BI // 2, 64, 0).astype(jnp.int32)
    proj = _project(emb_table.T, W, b.reshape(1, NUM_LABELS))
    summed = _pool(idx2, par, proj)
    return _head(summed)


# speedup vs baseline: 1.3197x; 1.3197x over previous
import functools

import jax
import jax.numpy as jnp
from jax import lax
from jax.experimental import pallas as pl
from jax.experimental.pallas import tpu as pltpu
from jax.experimental.pallas import tpu_sc as plsc

VOCAB = 1000000
BATCH = 4096
SEQ = 200
HIDDEN = 64
NUM_LABELS = 128

NUM_CORES = 2
NUM_SUBCORES = 16
NUM_WORKERS = NUM_CORES * NUM_SUBCORES
LANES = 16

ROWS_PER_WORKER = BATCH // NUM_WORKERS
CHUNK0 = 128
CHUNK1 = SEQ - CHUNK0


BI = 16384


def _project_body(t_ref, w_ref, b_ref, o_ref):
    o_ref[...] = (
        lax.dot_general(
            t_ref[...],
            w_ref[...],
            (((0,), (1,)), ((), ())),
            preferred_element_type=jnp.float32,
        )
        + b_ref[...]
    )


def _project(tableT, W, b2d):
    return pl.pallas_call(
        _project_body,
        grid=(pl.cdiv(VOCAB, BI),),
        in_specs=[
            pl.BlockSpec((HIDDEN, BI), lambda i: (0, i)),
            pl.BlockSpec((NUM_LABELS, HIDDEN), lambda i: (0, 0)),
            pl.BlockSpec((1, NUM_LABELS), lambda i: (0, 0)),
        ],
        out_specs=pl.BlockSpec((BI, NUM_LABELS), lambda i: (i, 0)),
        out_shape=jax.ShapeDtypeStruct((VOCAB, NUM_LABELS), jnp.float32),
    )(tableT, W, b2d)


def _pool_body(idx_hbm, table_hbm, out_hbm, idx_v, rows_v, out_v, sem0, sem1):
    wid = lax.axis_index("s") * NUM_CORES + lax.axis_index("c")
    base_row = wid * ROWS_PER_WORKER

    pltpu.sync_copy(
        idx_hbm.at[pl.ds(base_row * SEQ, ROWS_PER_WORKER * SEQ)], idx_v
    )

    sems = (sem0, sem1)

    def gather_descs(r, buf):
        off0 = pl.multiple_of(r * SEQ, 8)
        off1 = pl.multiple_of(r * SEQ + CHUNK0, 8)
        return (
            (
                table_hbm.at[idx_v.at[pl.ds(off0, CHUNK0)]],
                rows_v.at[buf].at[pl.ds(0, CHUNK0)],
                sems[buf],
            ),
            (
                table_hbm.at[idx_v.at[pl.ds(off1, CHUNK1)]],
                rows_v.at[buf].at[pl.ds(CHUNK0, CHUNK1)],
                sems[buf],
            ),
        )

    def start(r, buf):
        for desc in gather_descs(r, buf):
            pltpu.async_copy(*desc)

    def wait(r, buf):
        for desc in gather_descs(r, buf):
            pltpu.make_async_copy(*desc).wait()

    def accum(r, buf):
        def s_body(s, acc):
            return tuple(
                acc[j] + rows_v[buf, s, pl.ds(j * LANES, LANES)]
                for j in range(8)
            )

        zero = jnp.zeros((LANES,), jnp.float32)
        acc = lax.fori_loop(0, SEQ, s_body, (zero,) * 8)
        for j in range(8):
            out_v[r, pl.ds(j * LANES, LANES)] = acc[j]

    start(0, 0)

    def pair_body(k, carry):
        r0 = k * 2
        start(r0 + 1, 1)
        wait(r0, 0)
        accum(r0, 0)

        @pl.when(k < ROWS_PER_WORKER // 2 - 1)
        def _():
            start(r0 + 2, 0)

        wait(r0 + 1, 1)
        accum(r0 + 1, 1)
        return carry

    lax.fori_loop(0, ROWS_PER_WORKER // 2, pair_body, 0)
    pltpu.sync_copy(out_v, out_hbm.at[pl.ds(base_row, ROWS_PER_WORKER)])


_pool = pl.kernel(
    _pool_body,
    out_type=jax.ShapeDtypeStruct((BATCH, NUM_LABELS), jnp.float32),
    mesh=plsc.VectorSubcoreMesh(
        core_axis_name="c", subcore_axis_name="s", num_cores=NUM_CORES
    ),
    scratch_types=[
        pltpu.VMEM((ROWS_PER_WORKER * SEQ,), jnp.int32),
        pltpu.VMEM((2, SEQ, NUM_LABELS), jnp.float32),
        pltpu.VMEM((ROWS_PER_WORKER, NUM_LABELS), jnp.float32),
        pltpu.SemaphoreType.DMA,
        pltpu.SemaphoreType.DMA,
    ],
    compiler_params=pltpu.CompilerParams(use_tc_tiling_on_sc=True),
)


def _head_body(x_ref, o_ref):
    logits = x_ref[...] * (1.0 / SEQ)
    m = jnp.max(logits, axis=1, keepdims=True)
    e = jnp.exp(logits - m)
    s = jnp.sum(e, axis=1, keepdims=True)
    o_ref[...] = (logits - m) - jnp.log(s)


def _head(summed):
    return pl.pallas_call(
        _head_body,
        grid=(4,),
        in_specs=[pl.BlockSpec((BATCH // 4, NUM_LABELS), lambda i: (i, 0))],
        out_specs=pl.BlockSpec((BATCH // 4, NUM_LABELS), lambda i: (i, 0)),
        out_shape=jax.ShapeDtypeStruct((BATCH, NUM_LABELS), jnp.float32),
    )(summed)


@jax.jit
def kernel(one_hot_sentence, emb_table, W, b):
    idx = one_hot_sentence.reshape(-1).astype(jnp.int32)
    proj = _project(emb_table.T, W, b.reshape(1, NUM_LABELS))
    summed = _pool(idx, proj)
    return _head(summed)


# speedup vs baseline: 1.3284x; 1.0065x over previous
import functools

import jax
import jax.numpy as jnp
from jax import lax
from jax.experimental import pallas as pl
from jax.experimental.pallas import tpu as pltpu
from jax.experimental.pallas import tpu_sc as plsc

VOCAB = 1000000
BATCH = 4096
SEQ = 200
HIDDEN = 64
NUM_LABELS = 128

NUM_CORES = 2
NUM_SUBCORES = 16
NUM_WORKERS = NUM_CORES * NUM_SUBCORES
LANES = 16

ROWS_PER_WORKER = BATCH // NUM_WORKERS
CHUNK0 = 128
CHUNK1 = SEQ - CHUNK0


BI = 20480


def _project_body(t_ref, w_ref, b_ref, o_ref):
    o_ref[...] = (
        lax.dot_general(
            t_ref[...],
            w_ref[...],
            (((0,), (1,)), ((), ())),
            preferred_element_type=jnp.float32,
        )
        + b_ref[...]
    )


def _project(tableT, W, b2d):
    return pl.pallas_call(
        _project_body,
        grid=(pl.cdiv(VOCAB, BI),),
        in_specs=[
            pl.BlockSpec((HIDDEN, BI), lambda i: (0, i)),
            pl.BlockSpec((NUM_LABELS, HIDDEN), lambda i: (0, 0)),
            pl.BlockSpec((1, NUM_LABELS), lambda i: (0, 0)),
        ],
        out_specs=pl.BlockSpec((BI, NUM_LABELS), lambda i: (i, 0)),
        out_shape=jax.ShapeDtypeStruct((VOCAB, NUM_LABELS), jnp.float32),
    )(tableT, W, b2d)


def _pool_body(idx_hbm, table_hbm, out_hbm, idx_v, rows_v, out_v, sem0, sem1):
    wid = lax.axis_index("s") * NUM_CORES + lax.axis_index("c")
    base_row = wid * ROWS_PER_WORKER

    pltpu.sync_copy(
        idx_hbm.at[pl.ds(base_row * SEQ, ROWS_PER_WORKER * SEQ)], idx_v
    )

    sems = (sem0, sem1)

    def gather_descs(r, buf):
        off0 = pl.multiple_of(r * SEQ, 8)
        off1 = pl.multiple_of(r * SEQ + CHUNK0, 8)
        return (
            (
                table_hbm.at[idx_v.at[pl.ds(off0, CHUNK0)]],
                rows_v.at[buf].at[pl.ds(0, CHUNK0)],
                sems[buf],
            ),
            (
                table_hbm.at[idx_v.at[pl.ds(off1, CHUNK1)]],
                rows_v.at[buf].at[pl.ds(CHUNK0, CHUNK1)],
                sems[buf],
            ),
        )

    def start(r, buf):
        for desc in gather_descs(r, buf):
            pltpu.async_copy(*desc)

    def wait(r, buf):
        for desc in gather_descs(r, buf):
            pltpu.make_async_copy(*desc).wait()

    def accum(r, buf):
        def s_body(s, acc):
            return tuple(
                acc[j] + rows_v[buf, s, pl.ds(j * LANES, LANES)]
                for j in range(8)
            )

        zero = jnp.zeros((LANES,), jnp.float32)
        acc = lax.fori_loop(0, SEQ, s_body, (zero,) * 8)
        for j in range(8):
            out_v[r, pl.ds(j * LANES, LANES)] = acc[j]

    start(0, 0)

    def pair_body(k, carry):
        r0 = k * 2
        start(r0 + 1, 1)
        wait(r0, 0)
        accum(r0, 0)

        @pl.when(k < ROWS_PER_WORKER // 2 - 1)
        def _():
            start(r0 + 2, 0)

        wait(r0 + 1, 1)
        accum(r0 + 1, 1)
        return carry

    lax.fori_loop(0, ROWS_PER_WORKER // 2, pair_body, 0)
    pltpu.sync_copy(out_v, out_hbm.at[pl.ds(base_row, ROWS_PER_WORKER)])


_pool = pl.kernel(
    _pool_body,
    out_type=jax.ShapeDtypeStruct((BATCH, NUM_LABELS), jnp.float32),
    mesh=plsc.VectorSubcoreMesh(
        core_axis_name="c", subcore_axis_name="s", num_cores=NUM_CORES
    ),
    scratch_types=[
        pltpu.VMEM((ROWS_PER_WORKER * SEQ,), jnp.int32),
        pltpu.VMEM((2, SEQ, NUM_LABELS), jnp.float32),
        pltpu.VMEM((ROWS_PER_WORKER, NUM_LABELS), jnp.float32),
        pltpu.SemaphoreType.DMA,
        pltpu.SemaphoreType.DMA,
    ],
    compiler_params=pltpu.CompilerParams(use_tc_tiling_on_sc=True),
)


def _head_body(x_ref, o_ref):
    logits = x_ref[...] * (1.0 / SEQ)
    m = jnp.max(logits, axis=1, keepdims=True)
    e = jnp.exp(logits - m)
    s = jnp.sum(e, axis=1, keepdims=True)
    o_ref[...] = (logits - m) - jnp.log(s)


def _head(summed):
    return pl.pallas_call(
        _head_body,
        grid=(4,),
        in_specs=[pl.BlockSpec((BATCH // 4, NUM_LABELS), lambda i: (i, 0))],
        out_specs=pl.BlockSpec((BATCH // 4, NUM_LABELS), lambda i: (i, 0)),
        out_shape=jax.ShapeDtypeStruct((BATCH, NUM_LABELS), jnp.float32),
    )(summed)


@jax.jit
def kernel(one_hot_sentence, emb_table, W, b):
    idx = one_hot_sentence.reshape(-1).astype(jnp.int32)
    proj = _project(emb_table.T, W, b.reshape(1, NUM_LABELS))
    summed = _pool(idx, proj)
    return _head(summed)


# speedup vs baseline: 1.3288x; 1.0003x over previous
import functools

import jax
import jax.numpy as jnp
from jax import lax
from jax.experimental import pallas as pl
from jax.experimental.pallas import tpu as pltpu
from jax.experimental.pallas import tpu_sc as plsc

VOCAB = 1000000
BATCH = 4096
SEQ = 200
HIDDEN = 64
NUM_LABELS = 128

NUM_CORES = 2
NUM_SUBCORES = 16
NUM_WORKERS = NUM_CORES * NUM_SUBCORES
LANES = 16

ROWS_PER_WORKER = BATCH // NUM_WORKERS
CHUNKS = (56, 48, 48, 48)


BI = 20480


def _project_body(t_ref, w_ref, b_ref, o_ref):
    o_ref[...] = (
        lax.dot_general(
            t_ref[...],
            w_ref[...],
            (((0,), (1,)), ((), ())),
            preferred_element_type=jnp.float32,
        )
        + b_ref[...]
    )


def _project(tableT, W, b2d):
    return pl.pallas_call(
        _project_body,
        grid=(pl.cdiv(VOCAB, BI),),
        in_specs=[
            pl.BlockSpec((HIDDEN, BI), lambda i: (0, i)),
            pl.BlockSpec((NUM_LABELS, HIDDEN), lambda i: (0, 0)),
            pl.BlockSpec((1, NUM_LABELS), lambda i: (0, 0)),
        ],
        out_specs=pl.BlockSpec((BI, NUM_LABELS), lambda i: (i, 0)),
        out_shape=jax.ShapeDtypeStruct((VOCAB, NUM_LABELS), jnp.float32),
    )(tableT, W, b2d)


def _pool_body(idx_hbm, table_hbm, out_hbm, idx_v, rows_v, out_v, sem0, sem1):
    wid = lax.axis_index("s") * NUM_CORES + lax.axis_index("c")
    base_row = wid * ROWS_PER_WORKER

    pltpu.sync_copy(
        idx_hbm.at[pl.ds(base_row * SEQ, ROWS_PER_WORKER * SEQ)], idx_v
    )

    sems = (sem0, sem1)

    def gather_descs(r, buf):
        descs = []
        pos = 0
        for c in CHUNKS:
            off = pl.multiple_of(r * SEQ + pos, 8)
            descs.append((
                table_hbm.at[idx_v.at[pl.ds(off, c)]],
                rows_v.at[buf].at[pl.ds(pos, c)],
                sems[buf],
            ))
            pos += c
        return tuple(descs)

    def start(r, buf):
        for desc in gather_descs(r, buf):
            pltpu.async_copy(*desc)

    def wait(r, buf):
        for desc in gather_descs(r, buf):
            pltpu.make_async_copy(*desc).wait()

    def accum(r, buf):
        def s_body(s, acc):
            return tuple(
                acc[j] + rows_v[buf, s, pl.ds(j * LANES, LANES)]
                for j in range(8)
            )

        zero = jnp.zeros((LANES,), jnp.float32)
        acc = lax.fori_loop(0, SEQ, s_body, (zero,) * 8)
        for j in range(8):
            out_v[r, pl.ds(j * LANES, LANES)] = acc[j]

    start(0, 0)

    def pair_body(k, carry):
        r0 = k * 2
        start(r0 + 1, 1)
        wait(r0, 0)
        accum(r0, 0)

        @pl.when(k < ROWS_PER_WORKER // 2 - 1)
        def _():
            start(r0 + 2, 0)

        wait(r0 + 1, 1)
        accum(r0 + 1, 1)
        return carry

    lax.fori_loop(0, ROWS_PER_WORKER // 2, pair_body, 0)
    pltpu.sync_copy(out_v, out_hbm.at[pl.ds(base_row, ROWS_PER_WORKER)])


_pool = pl.kernel(
    _pool_body,
    out_type=jax.ShapeDtypeStruct((BATCH, NUM_LABELS), jnp.float32),
    mesh=plsc.VectorSubcoreMesh(
        core_axis_name="c", subcore_axis_name="s", num_cores=NUM_CORES
    ),
    scratch_types=[
        pltpu.VMEM((ROWS_PER_WORKER * SEQ,), jnp.int32),
        pltpu.VMEM((2, SEQ, NUM_LABELS), jnp.float32),
        pltpu.VMEM((ROWS_PER_WORKER, NUM_LABELS), jnp.float32),
        pltpu.SemaphoreType.DMA,
        pltpu.SemaphoreType.DMA,
    ],
    compiler_params=pltpu.CompilerParams(use_tc_tiling_on_sc=True),
)


def _head_body(x_ref, o_ref):
    logits = x_ref[...] * (1.0 / SEQ)
    m = jnp.max(logits, axis=1, keepdims=True)
    e = jnp.exp(logits - m)
    s = jnp.sum(e, axis=1, keepdims=True)
    o_ref[...] = (logits - m) - jnp.log(s)


def _head(summed):
    return pl.pallas_call(
        _head_body,
        grid=(4,),
        in_specs=[pl.BlockSpec((BATCH // 4, NUM_LABELS), lambda i: (i, 0))],
        out_specs=pl.BlockSpec((BATCH // 4, NUM_LABELS), lambda i: (i, 0)),
        out_shape=jax.ShapeDtypeStruct((BATCH, NUM_LABELS), jnp.float32),
    )(summed)


@jax.jit
def kernel(one_hot_sentence, emb_table, W, b):
    idx = one_hot_sentence.reshape(-1).astype(jnp.int32)
    proj = _project(emb_table.T, W, b.reshape(1, NUM_LABELS))
    summed = _pool(idx, proj)
    return _head(summed)


# speedup vs baseline: 1.3313x; 1.0019x over previous
import jax
import jax.numpy as jnp
from jax import lax
from jax.experimental import pallas as pl
from jax.experimental.pallas import tpu as pltpu
from jax.experimental.pallas import tpu_sc as plsc

VOCAB = 1000000
BATCH = 4096
SEQ = 200
HIDDEN = 64
NUM_LABELS = 128

NUM_CORES = 2
NUM_SUBCORES = 16
NUM_WORKERS = NUM_CORES * NUM_SUBCORES
LANES = 16

ROWS_PER_WORKER = BATCH // NUM_WORKERS
CHUNK0 = 128
CHUNK1 = SEQ - CHUNK0


BI = 20480


def _project_body(t_ref, w_ref, b_ref, o_ref):
    o_ref[...] = (
        lax.dot_general(
            t_ref[...],
            w_ref[...],
            (((0,), (1,)), ((), ())),
            preferred_element_type=jnp.float32,
        )
        + b_ref[...]
    )


def _project(tableT, W, b2d):
    return pl.pallas_call(
        _project_body,
        grid=(pl.cdiv(VOCAB, BI),),
        in_specs=[
            pl.BlockSpec((HIDDEN, BI), lambda i: (0, i)),
            pl.BlockSpec((NUM_LABELS, HIDDEN), lambda i: (0, 0)),
            pl.BlockSpec((1, NUM_LABELS), lambda i: (0, 0)),
        ],
        out_specs=pl.BlockSpec((BI, NUM_LABELS), lambda i: (i, 0)),
        out_shape=jax.ShapeDtypeStruct((VOCAB, NUM_LABELS), jnp.float32),
    )(tableT, W, b2d)


def _pool_body(idx_hbm, table_hbm, out_hbm, idx_v, rows_v, out_v, sem0, sem1):
    wid = lax.axis_index("s") * NUM_CORES + lax.axis_index("c")
    base_row = wid * ROWS_PER_WORKER

    pltpu.sync_copy(
        idx_hbm.at[pl.ds(base_row * SEQ, ROWS_PER_WORKER * SEQ)], idx_v
    )

    sems = (sem0, sem1)

    def gather_descs(r, buf):
        off0 = pl.multiple_of(r * SEQ, 8)
        off1 = pl.multiple_of(r * SEQ + CHUNK0, 8)
        return (
            (
                table_hbm.at[idx_v.at[pl.ds(off0, CHUNK0)]],
                rows_v.at[buf].at[pl.ds(0, CHUNK0)],
                sems[buf],
            ),
            (
                table_hbm.at[idx_v.at[pl.ds(off1, CHUNK1)]],
                rows_v.at[buf].at[pl.ds(CHUNK0, CHUNK1)],
                sems[buf],
            ),
        )

    def start(r, buf):
        for desc in gather_descs(r, buf):
            pltpu.async_copy(*desc)

    def wait(r, buf):
        for desc in gather_descs(r, buf):
            pltpu.make_async_copy(*desc).wait()

    def accum(r, buf):
        def s_body(s, acc):
            return tuple(
                acc[j] + rows_v[buf, s, pl.ds(j * LANES, LANES)]
                for j in range(8)
            )

        zero = jnp.zeros((LANES,), jnp.float32)
        acc = lax.fori_loop(0, SEQ, s_body, (zero,) * 8)
        for j in range(8):
            out_v[r, pl.ds(j * LANES, LANES)] = acc[j]

    start(0, 0)

    def pair_body(k, carry):
        r0 = k * 2
        start(r0 + 1, 1)
        wait(r0, 0)
        accum(r0, 0)

        @pl.when(k < ROWS_PER_WORKER // 2 - 1)
        def _():
            start(r0 + 2, 0)

        wait(r0 + 1, 1)
        accum(r0 + 1, 1)
        return carry

    lax.fori_loop(0, ROWS_PER_WORKER // 2, pair_body, 0)
    pltpu.sync_copy(out_v, out_hbm.at[pl.ds(base_row, ROWS_PER_WORKER)])


_pool = pl.kernel(
    _pool_body,
    out_type=jax.ShapeDtypeStruct((BATCH, NUM_LABELS), jnp.float32),
    mesh=plsc.VectorSubcoreMesh(
        core_axis_name="c", subcore_axis_name="s", num_cores=NUM_CORES
    ),
    scratch_types=[
        pltpu.VMEM((ROWS_PER_WORKER * SEQ,), jnp.int32),
        pltpu.VMEM((2, SEQ, NUM_LABELS), jnp.float32),
        pltpu.VMEM((ROWS_PER_WORKER, NUM_LABELS), jnp.float32),
        pltpu.SemaphoreType.DMA,
        pltpu.SemaphoreType.DMA,
    ],
    compiler_params=pltpu.CompilerParams(use_tc_tiling_on_sc=True),
)


def _head_body(x_ref, o_ref):
    logits = x_ref[...] * (1.0 / SEQ)
    m = jnp.max(logits, axis=1, keepdims=True)
    e = jnp.exp(logits - m)
    s = jnp.sum(e, axis=1, keepdims=True)
    o_ref[...] = (logits - m) - jnp.log(s)


def _head(summed):
    return pl.pallas_call(
        _head_body,
        grid=(4,),
        in_specs=[pl.BlockSpec((BATCH // 4, NUM_LABELS), lambda i: (i, 0))],
        out_specs=pl.BlockSpec((BATCH // 4, NUM_LABELS), lambda i: (i, 0)),
        out_shape=jax.ShapeDtypeStruct((BATCH, NUM_LABELS), jnp.float32),
    )(summed)


@jax.jit
def kernel(one_hot_sentence, emb_table, W, b):
    idx = one_hot_sentence.reshape(-1).astype(jnp.int32)
    proj = _project(emb_table.T, W, b.reshape(1, NUM_LABELS))
    summed = _pool(idx, proj)
    return _head(summed)
